# baseline (speedup 1.0000x reference)
import functools

import jax
import jax.numpy as jnp
from jax import lax
from jax.experimental import pallas as pl
from jax.experimental.pallas import tpu as pltpu
from jax.experimental.pallas import tpu_sc as plsc

NC = 2
NS = 16
L = 16
NW = NC * NS
NROWS = 32768
NCOLS = 1024

SC_ROWS = 0
ROWS_PER_W = SC_ROWS // NW
CHUNK_R = 32
NCHUNK = ROWS_PER_W // CHUNK_R
NACC = 8

TC_ROWS = NROWS - SC_ROWS
BR = 1024
NSTREAM = 4
SC_BLOCKS = SC_ROWS // BR
TC_STEPS = TC_ROWS // (BR * NSTREAM)


def _sc_body(x_hbm, out_hbm, buf0, buf1, accv, sem0, sem1):
    wid = lax.axis_index("s") * NC + lax.axis_index("c")
    row0 = wid * ROWS_PER_W
    bufs = (buf0, buf1)
    sems = (sem0, sem1)

    def copy(c, b):
        return pltpu.make_async_copy(
            x_hbm.at[pl.ds(row0 + c * CHUNK_R, CHUNK_R), :], bufs[b], sems[b])

    copy(0, 0).start()
    copy(1, 1).start()

    def sum_buf(buf, accs):
        def row_step(r, accs):
            new = list(accs)
            for u in range(NCOLS // L):
                v = buf[r, pl.ds(u * L, L)]
                new[u % NACC] = new[u % NACC] + jnp.maximum(v, 0.0)
            return tuple(new)
        return lax.fori_loop(0, CHUNK_R, row_step, accs)

    def body(c2, accs):
        c = c2 * 2
        copy(c, 0).wait()
        accs = sum_buf(buf0, accs)

        @pl.when(c2 < NCHUNK // 2 - 1)
        def _():
            copy(c + 2, 0).start()

        copy(c + 1, 1).wait()
        accs = sum_buf(buf1, accs)

        @pl.when(c2 < NCHUNK // 2 - 1)
        def _():
            copy(c + 3, 1).start()

        return accs

    accs = lax.fori_loop(
        0, NCHUNK // 2, body,
        tuple(jnp.zeros((L,), jnp.float32) for _ in range(NACC)))

    total = accs[0]
    for a in accs[1:]:
        total = total + a
    accv[...] = total
    pltpu.sync_copy(accv, out_hbm.at[pl.ds(wid * L, L)])


def _tc_body(*refs):
    x_refs = refs[:NSTREAM]
    out_ref, acc_ref = refs[NSTREAM], refs[NSTREAM + 1]
    i = pl.program_id(0)

    @pl.when(i == 0)
    def _():
        acc_ref[...] = jnp.zeros_like(acc_ref)

    acc = acc_ref[...]
    for x_ref in x_refs:
        for k in range(BR // 8):
            acc = acc + jnp.maximum(x_ref[pl.ds(8 * k, 8), :], 0.0)
    acc_ref[...] = acc

    @pl.when(i == pl.num_programs(0) - 1)
    def _():
        out_ref[0, 0] = jnp.sum(acc_ref[...])


def _sc_call(x):
    return pl.kernel(
        _sc_body,
        out_type=jax.ShapeDtypeStruct((NW * L,), jnp.float32),
        mesh=plsc.VectorSubcoreMesh(core_axis_name="c", subcore_axis_name="s"),
        scratch_types=[
            pltpu.VMEM((CHUNK_R, NCOLS), jnp.float32),
            pltpu.VMEM((CHUNK_R, NCOLS), jnp.float32),
            pltpu.VMEM((L,), jnp.float32),
            pltpu.SemaphoreType.DMA,
            pltpu.SemaphoreType.DMA,
        ],
        compiler_params=pltpu.CompilerParams(use_tc_tiling_on_sc=True),
    )(x)


def _tc_call(x):
    def make_index_map(j):
        return lambda i: (SC_BLOCKS + j * TC_STEPS + i, 0)

    return pl.pallas_call(
        _tc_body,
        grid=(TC_STEPS,),
        in_specs=[pl.BlockSpec((BR, NCOLS), make_index_map(j))
                  for j in range(NSTREAM)],
        out_specs=pl.BlockSpec(memory_space=pltpu.SMEM),
        out_shape=jax.ShapeDtypeStruct((1, 1), jnp.float32),
        scratch_shapes=[pltpu.VMEM((8, NCOLS), jnp.float32)],
    )(*([x] * NSTREAM))


def kernel(x):
    tc_sum = _tc_call(x)
    if SC_ROWS:
        sc_partials = _sc_call(x)
        return (jnp.sum(sc_partials) + tc_sum[0, 0])[None]
    return tc_sum[0, 0][None]


# speedup vs baseline: 1.0547x; 1.0547x over previous
import jax
import jax.numpy as jnp
from jax import lax
from jax.experimental import pallas as pl
from jax.experimental.pallas import tpu as pltpu

NROWS = 32768
NCOLS = 1024
CH_R = 512
NCH = NROWS // CH_R
NBUF = 4
UNR = 4


def _tc_body(x_hbm, out_ref, *bufs_and_sems):
    bufs = bufs_and_sems[:NBUF]
    sems = bufs_and_sems[NBUF:]

    def copy(c, b):
        return pltpu.make_async_copy(
            x_hbm.at[pl.ds(c * CH_R, CH_R), :], bufs[b], sems[b])

    for b in range(NBUF):
        copy(b, b).start()

    def outer(c4, acc):
        base = c4 * NBUF
        for b in range(NBUF):
            c = base + b
            copy(c, b).wait()
            buf = bufs[b]

            def inner(r, acc, buf=buf):
                a = acc
                for u in range(UNR):
                    a = a + jnp.maximum(buf[pl.ds((r * UNR + u) * 8, 8), :], 0.0)
                return a

            acc = lax.fori_loop(0, CH_R // (8 * UNR), inner, acc)

            @pl.when(c4 < NCH // NBUF - 1)
            def _():
                copy(c + NBUF, b).start()

        return acc

    acc = lax.fori_loop(0, NCH // NBUF, outer,
                        jnp.zeros((8, NCOLS), jnp.float32))
    out_ref[0, 0] = jnp.sum(acc)


def kernel(x):
    tc_sum = pl.pallas_call(
        _tc_body,
        in_specs=[pl.BlockSpec(memory_space=pltpu.HBM)],
        out_specs=pl.BlockSpec(memory_space=pltpu.SMEM),
        out_shape=jax.ShapeDtypeStruct((1, 1), jnp.float32),
        scratch_shapes=(
            [pltpu.VMEM((CH_R, NCOLS), jnp.float32) for _ in range(NBUF)]
            + [pltpu.SemaphoreType.DMA for _ in range(NBUF)]
        ),
    )(x)
    return tc_sum[0, 0][None]
